# Initial kernel scaffold; baseline (speedup 1.0000x reference)
import jax
import jax.numpy as jnp
from jax.experimental import pallas as pl


def kernel(x, u, z, edge_index, edge_weight, batch, batch_size, prev_h, Wxz, bxz, Whz, bhz, Wxr, bxr, Whr, bhr, Wxh, bxh, Whh, bhh, Wg, bg):
    raise NotImplementedError("write your pallas kernel here")


# speedup vs baseline: 5.4031x; 5.4031x over previous
import dataclasses
import functools

import jax
import jax.numpy as jnp
from jax import lax
from jax.experimental import pallas as pl
from jax.experimental.pallas import tpu as pltpu
from jax.experimental.pallas import tpu_sc as plsc

NC = 2
NS = 16
LANES = 16

F32 = jnp.float32


def _mesh():
    return plsc.VectorSubcoreMesh(core_axis_name="c", subcore_axis_name="s",
                                  num_cores=NC, num_subcores=NS)


def _sc_params():
    cp = pltpu.CompilerParams()
    if "needs_layout_passes" in pltpu.CompilerParams.__dataclass_fields__:
        cp = dataclasses.replace(cp, needs_layout_passes=False)
    return cp


def _build_norm_kernel(E, N):
    NPAD = ((N + NS * LANES - 1) // (NS * LANES)) * (NS * LANES)
    SL = NPAD // NS
    EPT = E // NS
    EPC = E // NC
    EPTN = EPC // NS
    BLK = 2000

    @functools.partial(
        pl.kernel,
        out_type=jax.ShapeDtypeStruct((E,), F32),
        mesh=_mesh(),
        compiler_params=_sc_params(),
        scratch_types=[
            pltpu.VMEM((NPAD,), F32),
            pltpu.VMEM((NPAD,), F32),
            pltpu.VMEM((BLK,), jnp.int32),
            pltpu.VMEM((BLK,), jnp.int32),
            pltpu.VMEM((BLK,), F32),
            pltpu.VMEM((BLK,), F32),
            pltpu.VMEM((SL,), F32),
            pltpu.VMEM((SL,), F32),
            pltpu.VMEM_SHARED((NS, NPAD), F32),
            pltpu.VMEM_SHARED((NPAD,), F32),
        ],
    )
    def knorm(src_hbm, dst_hbm, ew_hbm, norm_hbm, deg_v, dinv_v, sbuf, dbuf,
              wbuf, nbuf, tslice, accv, part_sh, dinv_sh):
        c = lax.axis_index("c")
        s = lax.axis_index("s")

        @pl.loop(0, NPAD, step=LANES)
        def _(i):
            deg_v[pl.ds(i, LANES)] = jnp.zeros((LANES,), F32)

        @pl.loop(0, EPT, step=BLK)
        def _(eo):
            base = s * EPT + eo
            pltpu.sync_copy(src_hbm.at[pl.ds(base, BLK)], sbuf)
            pltpu.sync_copy(ew_hbm.at[pl.ds(base, BLK)], wbuf)

            @pl.loop(0, BLK, step=LANES)
            def _(j):
                plsc.addupdate_scatter(
                    deg_v, [sbuf[pl.ds(j, LANES)]], wbuf[pl.ds(j, LANES)])

        pltpu.sync_copy(deg_v, part_sh.at[s])
        plsc.subcore_barrier()

        @pl.loop(0, SL, step=LANES)
        def _(i):
            accv[pl.ds(i, LANES)] = jnp.zeros((LANES,), F32)

        for t in range(NS):
            pltpu.sync_copy(part_sh.at[t, pl.ds(s * SL, SL)], tslice)

            @pl.loop(0, SL, step=LANES)
            def _(i):
                accv[pl.ds(i, LANES)] = (
                    accv[pl.ds(i, LANES)] + tslice[pl.ds(i, LANES)])

        @pl.loop(0, SL, step=LANES)
        def _(i):
            d = accv[pl.ds(i, LANES)]
            xc = jnp.maximum(d, 1e-12)
            ii = plsc.bitcast(xc, jnp.int32)
            ii = jnp.int32(0x5F3759DF) - lax.shift_right_arithmetic(ii, 1)
            y = plsc.bitcast(ii, F32)
            y = y * (1.5 - 0.5 * xc * y * y)
            y = y * (1.5 - 0.5 * xc * y * y)
            y = y * (1.5 - 0.5 * xc * y * y)
            y = y * (1.5 - 0.5 * xc * y * y)
            accv[pl.ds(i, LANES)] = jnp.where(d > 0, y, 0.0)

        pltpu.sync_copy(accv, dinv_sh.at[pl.ds(s * SL, SL)])
        plsc.subcore_barrier()
        pltpu.sync_copy(dinv_sh, dinv_v)

        @pl.loop(0, EPTN, step=BLK)
        def _(eo):
            base = c * EPC + s * EPTN + eo
            pltpu.sync_copy(src_hbm.at[pl.ds(base, BLK)], sbuf)
            pltpu.sync_copy(dst_hbm.at[pl.ds(base, BLK)], dbuf)
            pltpu.sync_copy(ew_hbm.at[pl.ds(base, BLK)], wbuf)

            @pl.loop(0, BLK, step=LANES)
            def _(j):
                a = plsc.load_gather(dinv_v, [sbuf[pl.ds(j, LANES)]])
                b = plsc.load_gather(dinv_v, [dbuf[pl.ds(j, LANES)]])
                nbuf[pl.ds(j, LANES)] = -(a * wbuf[pl.ds(j, LANES)] * b)

            pltpu.sync_copy(nbuf, norm_hbm.at[pl.ds(base, BLK)])

    return knorm


def _build_prop_kernel(E, N, D, nch):
    EPC = E // NC
    EPT = EPC // NS
    EB = 80
    ZR = 64
    NPAD = ((N + NS * ZR - 1) // (NS * ZR)) * (NS * ZR)
    RPT = NPAD // NS
    assert EPT % EB == 0 and RPT % ZR == 0

    @functools.partial(
        pl.kernel,
        out_type=jax.ShapeDtypeStruct((NC, nch, NPAD, D), F32),
        mesh=_mesh(),
        compiler_params=_sc_params(),
        scratch_types=[
            pltpu.VMEM((EB,), jnp.int32),
            pltpu.VMEM((EB,), jnp.int32),
            pltpu.VMEM((EB,), F32),
            pltpu.VMEM((EB, D), F32),
            pltpu.VMEM((ZR, D), F32),
            pltpu.VMEM_SHARED((NPAD, D), F32),
        ],
    )
    def kprop(*refs):
        tabs = refs[:nch]
        src_hbm, dst_hbm, nrm_hbm, out = refs[nch:nch + 4]
        sidx, didx, nrm, rows, zbuf, acc_sh = refs[nch + 4:]
        c = lax.axis_index("c")
        s = lax.axis_index("s")

        @pl.loop(0, ZR)
        def _(r):
            for q in range(D // LANES):
                zbuf[r, pl.ds(q * LANES, LANES)] = jnp.zeros((LANES,), F32)

        for ch in range(nch):
            tab = tabs[ch]

            @pl.loop(0, RPT, step=ZR)
            def _(k):
                pltpu.sync_copy(zbuf, acc_sh.at[pl.ds(s * RPT + k, ZR)])

            plsc.subcore_barrier()

            @pl.loop(0, EPT, step=EB)
            def _(eo):
                base = c * EPC + s * EPT + eo
                pltpu.sync_copy(src_hbm.at[pl.ds(base, EB)], sidx)
                pltpu.sync_copy(dst_hbm.at[pl.ds(base, EB)], didx)
                pltpu.sync_copy(nrm_hbm.at[pl.ds(base, EB)], nrm)
                pltpu.sync_copy(tab.at[sidx], rows)

                @pl.loop(0, EB, step=LANES)
                def _(j):
                    nv = nrm[pl.ds(j, LANES)]
                    for e in range(LANES):
                        w = nv[e]
                        for q in range(D // LANES):
                            rows[j + e, pl.ds(q * LANES, LANES)] = (
                                rows[j + e, pl.ds(q * LANES, LANES)] * w)

                pltpu.sync_copy(rows, acc_sh.at[didx], add=True)

            plsc.subcore_barrier()
            pltpu.sync_copy(acc_sh.at[pl.ds(s * RPT, RPT)],
                            out.at[c, ch, pl.ds(s * RPT, RPT)])
            if ch + 1 < nch:
                plsc.subcore_barrier()

    return kprop


def _gates_body(z_r, x_r, h_r, pz0_r, pz1_r, px0_r, px1_r, ph0_r, ph1_r,
                wx0z_r, wx0x_r, wx1z_r, wx1x_r, wh0_r, wh1_r, whh0_r,
                bx_r, bh_r, bhh_r, zg_r, hr_r, c1_r):
    zz = z_r[...]
    xx = x_r[...]
    hh = h_r[...]
    pz = pz0_r[...] + pz1_r[...]
    px = px0_r[...] + px1_r[...]
    ph = ph0_r[...] + ph1_r[...]
    dot = functools.partial(jnp.dot, preferred_element_type=F32)
    ax = (dot(zz, wx0z_r[...]) + dot(xx, wx0x_r[...])
          + dot(pz, wx1z_r[...]) + dot(px, wx1x_r[...]) + bx_r[...])
    ah = dot(hh, wh0_r[...]) + dot(ph, wh1_r[...]) + bh_r[...]
    zg = jax.nn.sigmoid(ax[:, :128] + ah[:, :128])
    rg = jax.nn.sigmoid(ax[:, 128:256] + ah[:, 128:])
    hr = hh * rg
    c1 = ax[:, 256:] + dot(hr, whh0_r[...]) + bhh_r[...]
    zg_r[...] = zg
    hr_r[...] = hr
    c1_r[...] = c1


def _gates_call(z, x, h, pz0, pz1, px0, px1, ph0, ph1,
                wx0z, wx0x, wx1z, wx1x, wh0, wh1, whh0, bx, bh, bhh):
    N, D = z.shape
    RB = 1000
    row = pl.BlockSpec((RB, D), lambda i: (i, 0))
    full = lambda a: pl.BlockSpec(a.shape, lambda i: (0,) * a.ndim)
    outs = [jax.ShapeDtypeStruct((N, D), F32)] * 3
    return pl.pallas_call(
        _gates_body,
        grid=(N // RB,),
        in_specs=[row] * 9 + [full(w) for w in
                              (wx0z, wx0x, wx1z, wx1x, wh0, wh1, whh0,
                               bx, bh, bhh)],
        out_specs=[row] * 3,
        out_shape=outs,
    )(z, x, h, pz0, pz1, px0, px1, ph0, ph1,
      wx0z, wx0x, wx1z, wx1x, wh0, wh1, whh0, bx, bh, bhh)


def _final_body(c1_r, zg_r, h_r, p0_r, p1_r, whh1_r, out_r):
    p = p0_r[...] + p1_r[...]
    ht = jnp.tanh(c1_r[...] + jnp.dot(p, whh1_r[...],
                                      preferred_element_type=F32))
    zg = zg_r[...]
    out_r[...] = zg * h_r[...] + (1.0 - zg) * ht


def _final_call(c1, zg, h, p0, p1, whh1):
    N, D = h.shape
    RB = 1000
    row = pl.BlockSpec((RB, D), lambda i: (i, 0))
    return pl.pallas_call(
        _final_body,
        grid=(N // RB,),
        in_specs=[row] * 5 + [pl.BlockSpec(whh1.shape, lambda i: (0, 0))],
        out_specs=row,
        out_shape=jax.ShapeDtypeStruct((N, D), F32),
    )(c1, zg, h, p0, p1, whh1)


def _readout_body(z_r, batch_r, u_r, wg_r, bg_r, out_r):
    zz = z_r[...]
    bb = batch_r[...]
    for b in range(out_r.shape[0]):
        mask = (bb == b)
        seg = jnp.sum(jnp.where(mask, zz, 0.0), axis=0)
        out_r[pl.ds(b, 1), pl.ds(0, zz.shape[1])] = seg[None, :]
    ge = jax.nn.relu(jnp.dot(u_r[...], wg_r[...],
                             preferred_element_type=F32) + bg_r[...])
    out_r[pl.ds(0, ge.shape[0]), pl.ds(zz.shape[1], ge.shape[1])] = ge


def _readout_call(z, batch2d, u, wg, bg2d):
    N, D = z.shape
    B, DG = u.shape
    full = lambda a: pl.BlockSpec(a.shape, lambda: (0,) * a.ndim)
    return pl.pallas_call(
        _readout_body,
        in_specs=[full(z), full(batch2d), full(u), full(wg), full(bg2d)],
        out_specs=pl.BlockSpec((B, 2 * D), lambda: (0, 0)),
        out_shape=jax.ShapeDtypeStruct((B, 2 * D), F32),
    )(z, batch2d, u, wg, bg2d)


def kernel(x, u, z, edge_index, edge_weight, batch, batch_size, prev_h,
           Wxz, bxz, Whz, bhz, Wxr, bxr, Whr, bhr, Wxh, bxh, Whh, bhh,
           Wg, bg):
    N, DN = x.shape
    E = edge_index.shape[1]
    D = z.shape[1]
    src = edge_index[0]
    dst = edge_index[1]

    norm = _build_norm_kernel(E, N)(src, dst, edge_weight)

    pout = _build_prop_kernel(E, N, D, 3)(z, x, prev_h, src, dst, norm)
    pout = pout[:, :, :N]

    wx0 = jnp.concatenate([Wxz[0], Wxr[0], Wxh[0]], axis=1)
    wx1 = jnp.concatenate([Wxz[1], Wxr[1], Wxh[1]], axis=1)
    wh0 = jnp.concatenate([Whz[0], Whr[0]], axis=1)
    wh1 = jnp.concatenate([Whz[1], Whr[1]], axis=1)
    bx = jnp.concatenate([bxz, bxr, bxh])[None, :]
    bh = jnp.concatenate([bhz, bhr])[None, :]

    zg, hr, c1 = _gates_call(
        z, x, prev_h,
        pout[0, 0], pout[1, 0], pout[0, 1], pout[1, 1], pout[0, 2],
        pout[1, 2],
        wx0[:D], wx0[D:], wx1[:D], wx1[D:], wh0, wh1, Whh[0],
        bx, bh, bhh[None, :])

    phr = _build_prop_kernel(E, N, D, 1)(hr, src, dst, norm)[:, :, :N]
    h_new = _final_call(c1, zg, prev_h, phr[0, 0], phr[1, 0], Whh[1])

    fused = _readout_call(z, batch[:, None], u, Wg, bg[None, :])
    return (fused, h_new)
